# baseline (speedup 1.0000x reference)
import functools
import jax
import jax.numpy as jnp
from jax import lax
from jax.experimental import pallas as pl
from jax.experimental.pallas import tpu as pltpu
from jax.experimental.pallas import tpu_sc as plsc

D = 64
NC = 2
NS = 16
NW = NC * NS
SUB = 128
KSUB = 4
CHUNK = SUB * KSUB


@functools.cache
def _make_kernel(b_total):
    assert b_total % (NW * CHUNK) == 0
    b_per_w = b_total // NW
    n_chunks = b_per_w // CHUNK
    mesh = plsc.VectorSubcoreMesh(
        core_axis_name="c", subcore_axis_name="s",
        num_cores=NC, num_subcores=NS)

    @functools.partial(
        pl.kernel,
        out_type=jax.ShapeDtypeStruct((b_total // SUB, SUB, D), jnp.float32),
        mesh=mesh,
        scratch_types=[
            pltpu.VMEM((KSUB, SUB), jnp.int32),
            pltpu.VMEM((KSUB, SUB, D), jnp.float32),
            pltpu.SemaphoreType.DMA,
        ],
        compiler_params=pltpu.CompilerParams(use_tc_tiling_on_sc=False),
    )
    def gather_kernel(idx_hbm, table_hbm, out_hbm, idx_v, rows_v, sem):
        wid = lax.axis_index("s") * NC + lax.axis_index("c")
        row0 = wid * (b_per_w // SUB)

        @pl.loop(0, n_chunks)
        def chunk(ci):
            r = row0 + ci * KSUB
            pltpu.sync_copy(idx_hbm.at[pl.ds(r, KSUB)], idx_v)
            copies = [
                pltpu.async_copy(table_hbm.at[idx_v.at[j]], rows_v.at[j], sem)
                for j in range(KSUB)
            ]
            for c in copies:
                c.wait()
            pltpu.sync_copy(rows_v, out_hbm.at[pl.ds(r, KSUB)])

    return gather_kernel


def kernel(nodes, table):
    b_total = nodes.shape[0] * nodes.shape[1]
    idx = nodes.reshape(b_total // SUB, SUB)
    out = _make_kernel(b_total)(idx, table)
    return out.reshape(nodes.shape[0], nodes.shape[1], D)


# speedup vs baseline: 1.0210x; 1.0210x over previous
import functools
import jax
import jax.numpy as jnp
from jax import lax
from jax.experimental import pallas as pl
from jax.experimental.pallas import tpu as pltpu
from jax.experimental.pallas import tpu_sc as plsc

D = 64
NC = 2
NS = 16
NW = NC * NS
SUB = 128
KSUB = 4
CHUNK = SUB * KSUB


@functools.cache
def _make_kernel(b_total):
    assert b_total % (NW * CHUNK * 2) == 0
    b_per_w = b_total // NW
    n_chunks = b_per_w // CHUNK
    mesh = plsc.VectorSubcoreMesh(
        core_axis_name="c", subcore_axis_name="s",
        num_cores=NC, num_subcores=NS)

    @functools.partial(
        pl.kernel,
        out_type=jax.ShapeDtypeStruct((b_total // SUB, SUB, D), jnp.float32),
        mesh=mesh,
        scratch_types=[
            pltpu.VMEM((2, KSUB, SUB), jnp.int32),
            pltpu.VMEM((2, KSUB, SUB, D), jnp.float32),
            pltpu.SemaphoreType.DMA,
            pltpu.SemaphoreType.DMA,
            pltpu.SemaphoreType.DMA,
        ],
        compiler_params=pltpu.CompilerParams(use_tc_tiling_on_sc=False),
    )
    def gather_kernel(idx_hbm, table_hbm, out_hbm, idx_v, rows_v,
                      gsem, osem0, osem1):
        wid = lax.axis_index("s") * NC + lax.axis_index("c")
        row0 = wid * (b_per_w // SUB)
        osem = (osem0, osem1)

        def gather_chunk(g, b):
            pltpu.sync_copy(idx_hbm.at[pl.ds(row0 + g * KSUB, KSUB)],
                            idx_v.at[b])
            copies = [
                pltpu.async_copy(table_hbm.at[idx_v.at[b].at[j]],
                                 rows_v.at[b].at[j], gsem)
                for j in range(KSUB)
            ]
            for c in copies:
                c.wait()

        def start_out(g, b):
            pltpu.async_copy(rows_v.at[b],
                             out_hbm.at[pl.ds(row0 + g * KSUB, KSUB)],
                             osem[b])

        def wait_out(b):
            pltpu.make_async_copy(
                rows_v.at[b], out_hbm.at[pl.ds(row0, KSUB)], osem[b]).wait()

        gather_chunk(0, 0)
        start_out(0, 0)
        gather_chunk(1, 1)
        start_out(1, 1)

        @pl.loop(1, n_chunks // 2)
        def body(s):
            g = s * 2
            wait_out(0)
            gather_chunk(g, 0)
            start_out(g, 0)
            wait_out(1)
            gather_chunk(g + 1, 1)
            start_out(g + 1, 1)

        wait_out(0)
        wait_out(1)

    return gather_kernel


def kernel(nodes, table):
    b_total = nodes.shape[0] * nodes.shape[1]
    idx = nodes.reshape(b_total // SUB, SUB)
    out = _make_kernel(b_total)(idx, table)
    return out.reshape(nodes.shape[0], nodes.shape[1], D)


# speedup vs baseline: 1.0269x; 1.0058x over previous
import functools
import jax
import jax.numpy as jnp
from jax import lax
from jax.experimental import pallas as pl
from jax.experimental.pallas import tpu as pltpu
from jax.experimental.pallas import tpu_sc as plsc

D = 64
NC = 2
NS = 16
NW = NC * NS
CHUNK = 128
NBUF = 4


@functools.cache
def _make_kernel(n_lookups):
    n_chunks = n_lookups // CHUNK
    assert n_lookups % CHUNK == 0 and n_chunks % NW == 0
    cpw = n_chunks // NW
    assert cpw % NBUF == 0
    nsteps = cpw // NBUF
    mesh = plsc.VectorSubcoreMesh(
        core_axis_name="c", subcore_axis_name="s",
        num_cores=NC, num_subcores=NS)

    @functools.partial(
        pl.kernel,
        out_type=jax.ShapeDtypeStruct((n_lookups, D), jnp.float32),
        mesh=mesh,
        scratch_types=(
            [pltpu.VMEM((cpw, CHUNK), jnp.int32)]
            + [pltpu.VMEM((CHUNK, D), jnp.float32) for _ in range(NBUF)]
            + [pltpu.SemaphoreType.DMA for _ in range(2 * NBUF)]
        ),
        compiler_params=pltpu.CompilerParams(use_tc_tiling_on_sc=False),
    )
    def gather_kernel(idx_hbm, table_hbm, out_hbm, idx_all, *bufs_sems):
        rows = bufs_sems[:NBUF]
        gsem = bufs_sems[NBUF:2 * NBUF]
        osem = bufs_sems[2 * NBUF:]
        wid = lax.axis_index("s") * NC + lax.axis_index("c")
        chunk0 = wid * cpw

        pltpu.sync_copy(idx_hbm.at[pl.ds(chunk0, cpw)], idx_all)

        def fire(i, b):
            pltpu.async_copy(table_hbm.at[idx_all.at[i]], rows[b], gsem[b])

        def wait_gather(b):
            pltpu.make_async_copy(table_hbm.at[idx_all.at[0]],
                                  rows[b], gsem[b]).wait()

        def wb_start(i, b):
            pltpu.async_copy(rows[b],
                             out_hbm.at[pl.ds((chunk0 + i) * CHUNK, CHUNK)],
                             osem[b])

        def wait_wb(b):
            pltpu.make_async_copy(rows[b], out_hbm.at[pl.ds(0, CHUNK)],
                                  osem[b]).wait()

        for b in range(NBUF):
            fire(b, b)

        @pl.loop(0, nsteps)
        def body(s):
            i0 = s * NBUF
            for b in range(NBUF):
                wait_gather(b)
                wb_start(i0 + b, b)

            @pl.when(s < nsteps - 1)
            def _():
                for b in range(NBUF):
                    wait_wb(b)
                    fire(i0 + NBUF + b, b)

        for b in range(NBUF):
            wait_wb(b)

    return gather_kernel


def kernel(nodes, table):
    batch, n_fields = nodes.shape
    n_lookups = batch * n_fields
    idx = nodes.reshape(n_lookups // CHUNK, CHUNK)
    out = _make_kernel(n_lookups)(idx, table)
    return out.reshape(batch, n_fields, D)


# speedup vs baseline: 1.0279x; 1.0009x over previous
import functools
import jax
import jax.numpy as jnp
from jax import lax
from jax.experimental import pallas as pl
from jax.experimental.pallas import tpu as pltpu
from jax.experimental.pallas import tpu_sc as plsc

D = 64
NC = 2
NS = 16
NW = NC * NS
CHUNK = 128
NBUF = 8


@functools.cache
def _make_kernel(n_lookups):
    n_chunks = n_lookups // CHUNK
    assert n_lookups % CHUNK == 0 and n_chunks % NW == 0
    cpw = n_chunks // NW
    assert cpw % NBUF == 0
    nsteps = cpw // NBUF
    mesh = plsc.VectorSubcoreMesh(
        core_axis_name="c", subcore_axis_name="s",
        num_cores=NC, num_subcores=NS)

    @functools.partial(
        pl.kernel,
        out_type=jax.ShapeDtypeStruct((n_lookups, D), jnp.float32),
        mesh=mesh,
        scratch_types=(
            [pltpu.VMEM((cpw, CHUNK), jnp.int32)]
            + [pltpu.VMEM((CHUNK, D), jnp.float32) for _ in range(NBUF)]
            + [pltpu.SemaphoreType.DMA for _ in range(2 * NBUF)]
        ),
        compiler_params=pltpu.CompilerParams(use_tc_tiling_on_sc=False),
    )
    def gather_kernel(idx_hbm, table_hbm, out_hbm, idx_all, *bufs_sems):
        rows = bufs_sems[:NBUF]
        gsem = bufs_sems[NBUF:2 * NBUF]
        osem = bufs_sems[2 * NBUF:]
        wid = lax.axis_index("s") * NC + lax.axis_index("c")
        chunk0 = wid * cpw

        pltpu.sync_copy(idx_hbm.at[pl.ds(chunk0, cpw)], idx_all)

        def fire(i, b):
            pltpu.async_copy(table_hbm.at[idx_all.at[i]], rows[b], gsem[b])

        def wait_gather(b):
            pltpu.make_async_copy(table_hbm.at[idx_all.at[0]],
                                  rows[b], gsem[b]).wait()

        def wb_start(i, b):
            pltpu.async_copy(rows[b],
                             out_hbm.at[pl.ds((chunk0 + i) * CHUNK, CHUNK)],
                             osem[b])

        def wait_wb(b):
            pltpu.make_async_copy(rows[b], out_hbm.at[pl.ds(0, CHUNK)],
                                  osem[b]).wait()

        for b in range(NBUF):
            fire(b, b)

        @pl.loop(0, nsteps)
        def body(s):
            i0 = s * NBUF
            for b in range(NBUF):
                wait_gather(b)
                wb_start(i0 + b, b)

            @pl.when(s < nsteps - 1)
            def _():
                for b in range(NBUF):
                    wait_wb(b)
                    fire(i0 + NBUF + b, b)

        for b in range(NBUF):
            wait_wb(b)

    return gather_kernel


def kernel(nodes, table):
    batch, n_fields = nodes.shape
    n_lookups = batch * n_fields
    idx = nodes.reshape(n_lookups // CHUNK, CHUNK)
    out = _make_kernel(n_lookups)(idx, table)
    return out.reshape(batch, n_fields, D)


# speedup vs baseline: 1.1139x; 1.0837x over previous
import functools
import jax
import jax.numpy as jnp
from jax import lax
from jax.experimental import pallas as pl
from jax.experimental.pallas import tpu as pltpu
from jax.experimental.pallas import tpu_sc as plsc

D = 64
NC = 2
NS = 16
NW = NC * NS
CHUNK = 128
NBUF = 8


@functools.cache
def _make_kernel(n_lookups):
    n_chunks = n_lookups // CHUNK
    assert n_lookups % CHUNK == 0 and n_chunks % NW == 0
    cpw = n_chunks // NW
    assert cpw % NBUF == 0
    nsteps = cpw // NBUF
    mesh = plsc.VectorSubcoreMesh(
        core_axis_name="c", subcore_axis_name="s",
        num_cores=NC, num_subcores=NS)

    @functools.partial(
        pl.kernel,
        out_type=jax.ShapeDtypeStruct((n_lookups, D), jnp.float32),
        mesh=mesh,
        scratch_types=(
            [pltpu.VMEM((cpw, CHUNK), jnp.int32)]
            + [pltpu.VMEM((CHUNK, D), jnp.float32) for _ in range(NBUF)]
            + [pltpu.SemaphoreType.DMA for _ in range(2 * NBUF)]
        ),
        compiler_params=pltpu.CompilerParams(use_tc_tiling_on_sc=False),
    )
    def gather_kernel(idx_hbm, table_hbm, out_hbm, idx_all, *bufs_sems):
        rows = bufs_sems[:NBUF]
        gsem = bufs_sems[NBUF:2 * NBUF]
        osem = bufs_sems[2 * NBUF:]
        wid = lax.axis_index("s") * NC + lax.axis_index("c")
        chunk0 = wid * cpw

        pltpu.sync_copy(idx_hbm.at[pl.ds(chunk0, cpw)], idx_all)

        def fire(i, b):
            pltpu.async_copy(table_hbm.at[idx_all.at[i]], rows[b], gsem[b])

        def wait_gather(b):
            pltpu.make_async_copy(table_hbm.at[idx_all.at[0]],
                                  rows[b], gsem[b]).wait()

        def wb_start(i, b):
            pltpu.async_copy(rows[b],
                             out_hbm.at[pl.ds((chunk0 + i) * CHUNK, CHUNK)],
                             osem[b])

        def wait_wb(b):
            pltpu.make_async_copy(rows[b], out_hbm.at[pl.ds(0, CHUNK)],
                                  osem[b]).wait()

        for b in range(NBUF):
            fire(b, b)

        @pl.loop(0, nsteps)
        def body(s):
            i0 = s * NBUF
            for b in range(NBUF):
                wait_gather(b)
                wb_start(i0 + b, b)

            @pl.when(s < nsteps - 1)
            def _():
                for b in range(NBUF):
                    wait_wb(b)
                    fire(i0 + NBUF + b, b)

        for b in range(NBUF):
            wait_wb(b)

    return gather_kernel


@functools.cache
def _make_tc_transpose(batch, n_fields):
    bh = batch // CHUNK
    fp = n_fields // 2

    def body(x_ref, y_ref):
        x4 = x_ref[0].reshape(CHUNK, fp, 2, D)
        for f in range(n_fields):
            blk = x4[:, f // 2, f % 2, :]
            y_ref[f, :, 0, :, :] = blk.T.reshape(D // 8, 8, CHUNK)

    return pl.pallas_call(
        body,
        grid=(bh,),
        in_specs=[pl.BlockSpec((1, n_fields * D, CHUNK),
                               lambda i: (i, 0, 0))],
        out_specs=pl.BlockSpec((n_fields, D // 8, 1, 8, CHUNK),
                               lambda i: (0, 0, i, 0, 0)),
        out_shape=jax.ShapeDtypeStruct(
            (n_fields, D // 8, bh, 8, CHUNK), jnp.float32),
    )


def kernel(nodes, table):
    batch, n_fields = nodes.shape
    n_lookups = batch * n_fields
    idx = nodes.reshape(n_lookups // CHUNK, CHUNK)
    out = _make_kernel(n_lookups)(idx, table)
    x3 = out.reshape(batch // CHUNK, n_fields * D, CHUNK)
    out5 = _make_tc_transpose(batch, n_fields)(x3)
    return out5.transpose(2, 4, 0, 1, 3).reshape(batch, n_fields, D)


# speedup vs baseline: 1.4127x; 1.2683x over previous
import functools
import jax
import jax.numpy as jnp
from jax import lax
from jax.experimental import pallas as pl
from jax.experimental.pallas import tpu as pltpu
from jax.experimental.pallas import tpu_sc as plsc

D = 64
NC = 2
NS = 16
NW = NC * NS
CHUNK = 128
NBUF = 8


@functools.cache
def _make_kernel(n_lookups):
    n_chunks = n_lookups // CHUNK
    assert n_lookups % CHUNK == 0 and n_chunks % NW == 0
    cpw = n_chunks // NW
    assert cpw % NBUF == 0
    nsteps = cpw // NBUF
    mesh = plsc.VectorSubcoreMesh(
        core_axis_name="c", subcore_axis_name="s",
        num_cores=NC, num_subcores=NS)

    @functools.partial(
        pl.kernel,
        out_type=jax.ShapeDtypeStruct((n_lookups, D), jnp.float32),
        mesh=mesh,
        scratch_types=(
            [pltpu.VMEM((cpw, CHUNK), jnp.int32)]
            + [pltpu.VMEM((CHUNK, D), jnp.float32) for _ in range(NBUF)]
            + [pltpu.SemaphoreType.DMA for _ in range(2 * NBUF)]
        ),
        compiler_params=pltpu.CompilerParams(use_tc_tiling_on_sc=False),
    )
    def gather_kernel(idx_hbm, table_hbm, out_hbm, idx_all, *bufs_sems):
        rows = bufs_sems[:NBUF]
        gsem = bufs_sems[NBUF:2 * NBUF]
        osem = bufs_sems[2 * NBUF:]
        wid = lax.axis_index("s") * NC + lax.axis_index("c")
        chunk0 = wid * cpw

        pltpu.sync_copy(idx_hbm.at[pl.ds(chunk0, cpw)], idx_all)

        def fire(i, b):
            pltpu.async_copy(table_hbm.at[idx_all.at[i]], rows[b], gsem[b])

        def wait_gather(b):
            pltpu.make_async_copy(table_hbm.at[idx_all.at[0]],
                                  rows[b], gsem[b]).wait()

        def wb_start(i, b):
            pltpu.async_copy(rows[b],
                             out_hbm.at[pl.ds((chunk0 + i) * CHUNK, CHUNK)],
                             osem[b])

        def wait_wb(b):
            pltpu.make_async_copy(rows[b], out_hbm.at[pl.ds(0, CHUNK)],
                                  osem[b]).wait()

        for b in range(NBUF):
            fire(b, b)

        @pl.loop(0, nsteps)
        def body(s):
            i0 = s * NBUF
            for b in range(NBUF):
                wait_gather(b)
                wb_start(i0 + b, b)

            @pl.when(s < nsteps - 1)
            def _():
                for b in range(NBUF):
                    wait_wb(b)
                    fire(i0 + NBUF + b, b)

        for b in range(NBUF):
            wait_wb(b)

    return gather_kernel


@functools.cache
def _make_tc_transpose(batch, n_fields):
    bh = batch // CHUNK
    fp = n_fields // 2

    def body(x_ref, y_ref):
        x4 = x_ref[0].reshape(CHUNK, fp, 2, D)
        for f in range(n_fields):
            blk = x4[:, f // 2, f % 2, :]
            y_ref[f, :, 0, :, :] = blk.T.reshape(D // 8, 8, CHUNK)

    return pl.pallas_call(
        body,
        grid=(bh,),
        in_specs=[pl.BlockSpec((1, n_fields * D, CHUNK),
                               lambda i: (i, 0, 0))],
        out_specs=pl.BlockSpec((n_fields, D // 8, 1, 8, CHUNK),
                               lambda i: (0, 0, i, 0, 0)),
        out_shape=jax.ShapeDtypeStruct(
            (n_fields, D // 8, bh, 8, CHUNK), jnp.float32),
    )


@functools.cache
def _make_tc_repack(vocab, bv):
    def body(x_ref, y_ref):
        xt = x_ref[...].T.reshape(bv // 2, 2, D)
        y_ref[...] = jnp.concatenate([xt[:, 0, :], xt[:, 1, :]], axis=1)

    return pl.pallas_call(
        body,
        grid=(pl.cdiv(vocab, bv),),
        in_specs=[pl.BlockSpec((D, bv), lambda i: (0, i))],
        out_specs=pl.BlockSpec((bv // 2, 2 * D), lambda i: (i, 0)),
        out_shape=jax.ShapeDtypeStruct((vocab // 2, 2 * D), jnp.float32),
    )


def kernel(nodes, table):
    batch, n_fields = nodes.shape
    vocab = table.shape[0]
    n_lookups = batch * n_fields
    idx = nodes.reshape(n_lookups // CHUNK, CHUNK)
    packed = _make_tc_repack(vocab, 8192)(table.T)
    tlin = packed.reshape(vocab, D)
    out = _make_kernel(n_lookups)(idx, tlin)
    x3 = out.reshape(batch // CHUNK, n_fields * D, CHUNK)
    out5 = _make_tc_transpose(batch, n_fields)(x3)
    return out5.transpose(2, 4, 0, 1, 3).reshape(batch, n_fields, D)
